# baseline (speedup 1.0000x reference)
import functools
import jax
import jax.numpy as jnp
from jax import lax
from jax.experimental import pallas as pl
from jax.experimental.pallas import tpu as pltpu
from jax.experimental.pallas import tpu_sc as plsc

N1, E, D, H = 10000, 160000, 256, 128
NUM_CLASSES = 10
K1, K2 = 8000, 6400
NP1, NP2, NP3 = 10240, 8192, 6656
EPAD = 163840
EPW = EPAD // 32
CH = 128
IMIN = -2147483648

_mesh = plsc.VectorSubcoreMesh(core_axis_name="c", subcore_axis_name="s")


def _key_from_score(s, rows, n_real):
    b = lax.bitcast_convert_type(s, jnp.int32)
    b = jnp.where(b == IMIN, 0, b)
    key = jnp.where(b < 0, b ^ 0x7FFFFFFF, b)
    return jnp.where(rows < n_real, key, IMIN)


def _score_tail(h, w, br, n_real, hs_ref, key_ref):
    wn = jnp.sqrt(jnp.sum(w * w))
    arg = lax.dot_general(h, w, (((1,), (0,)), ((), ())),
                          preferred_element_type=jnp.float32) / wn
    s = jnp.tanh(arg)
    hs_ref[...] = h * s
    rows = pl.program_id(0) * br + lax.broadcasted_iota(jnp.int32, (br, 1), 0)
    key_ref[...] = _key_from_score(s, rows, n_real)


_DN = (((1,), (1,)), ((), ()))


def _combine1_body(n_real, br, p_ref, x_ref, wrel_ref, wroot_ref,
                   ba_ref, bb_ref, w_ref, hs_ref, key_ref):
    a = lax.dot_general(p_ref[...], wrel_ref[...], _DN,
                        preferred_element_type=jnp.float32)
    xr = lax.dot_general(x_ref[...], wroot_ref[...], _DN,
                         preferred_element_type=jnp.float32)
    h = jnp.maximum((a + ba_ref[...]) + (xr + bb_ref[...]), 0.0)
    _score_tail(h, w_ref[...], br, n_real, hs_ref, key_ref)


def _combine1(npad, n_real, br):
    return pl.pallas_call(
        functools.partial(_combine1_body, n_real, br),
        grid=(npad // br,),
        in_specs=[
            pl.BlockSpec((br, D), lambda i: (i, 0)),
            pl.BlockSpec((br, D), lambda i: (i, 0)),
            pl.BlockSpec((H, D), lambda i: (0, 0)),
            pl.BlockSpec((H, D), lambda i: (0, 0)),
            pl.BlockSpec((1, H), lambda i: (0, 0)),
            pl.BlockSpec((1, H), lambda i: (0, 0)),
            pl.BlockSpec((H, 1), lambda i: (0, 0)),
        ],
        out_specs=[
            pl.BlockSpec((br, H), lambda i: (i, 0)),
            pl.BlockSpec((br, 1), lambda i: (i, 0)),
        ],
        out_shape=[
            jax.ShapeDtypeStruct((npad, H), jnp.float32),
            jax.ShapeDtypeStruct((npad, 1), jnp.int32),
        ],
    )


def _combine23_body(n_real, br, p_ref, hin_ref, wrel_ref, wroot_ref,
                    ba_ref, bb_ref, w_ref, hs_ref, key_ref):
    a = lax.dot_general(p_ref[...], wrel_ref[...], _DN,
                        preferred_element_type=jnp.float32)
    xr = lax.dot_general(hin_ref[...], wroot_ref[...], _DN,
                         preferred_element_type=jnp.float32)
    h = jnp.maximum((a + ba_ref[...]) + (xr + bb_ref[...]), 0.0)
    _score_tail(h, w_ref[...], br, n_real, hs_ref, key_ref)


def _combine23(npad, n_real, br):
    return pl.pallas_call(
        functools.partial(_combine23_body, n_real, br),
        grid=(npad // br,),
        in_specs=[
            pl.BlockSpec((br, H), lambda i: (i, 0)),
            pl.BlockSpec((br, H), lambda i: (i, 0)),
            pl.BlockSpec((H, H), lambda i: (0, 0)),
            pl.BlockSpec((H, H), lambda i: (0, 0)),
            pl.BlockSpec((1, H), lambda i: (0, 0)),
            pl.BlockSpec((1, H), lambda i: (0, 0)),
            pl.BlockSpec((H, 1), lambda i: (0, 0)),
        ],
        out_specs=[
            pl.BlockSpec((br, H), lambda i: (i, 0)),
            pl.BlockSpec((br, 1), lambda i: (i, 0)),
        ],
        out_shape=[
            jax.ShapeDtypeStruct((npad, H), jnp.float32),
            jax.ShapeDtypeStruct((npad, 1), jnp.int32),
        ],
    )


def _rank_body(cchunks, ki_ref, kt_ref, rank_ref):
    bi = pl.program_id(0)
    ki = ki_ref[0]
    iglob = bi * 128 + lax.broadcasted_iota(jnp.int32, (1, 128), 1)

    def step(c, acc):
        kj = kt_ref[c]
        jglob = c * 32 + lax.broadcasted_iota(jnp.int32, (32, 1), 0)
        jlt = jglob < iglob
        ge = jnp.where(kj >= ki, jnp.int32(1), jnp.int32(0))
        gt = jnp.where(kj > ki, jnp.int32(1), jnp.int32(0))
        return acc + jnp.where(jlt, ge, gt)

    acc = lax.fori_loop(0, cchunks, step, jnp.zeros((32, 128), jnp.int32))
    rank_ref[0] = jnp.sum(acc, axis=0, keepdims=True)


def _rank(npad):
    rrows, cchunks = npad // 128, npad // 32
    return pl.pallas_call(
        functools.partial(_rank_body, cchunks),
        grid=(rrows,),
        in_specs=[
            pl.BlockSpec((1, 1, 128), lambda i: (i, 0, 0)),
            pl.BlockSpec((cchunks, 32, 1), lambda i: (0, 0, 0)),
        ],
        out_specs=pl.BlockSpec((1, 1, 128), lambda i: (i, 0, 0)),
        out_shape=jax.ShapeDtypeStruct((rrows, 1, 128), jnp.int32),
    )


CHE = 4096
PEND = 256
FL = 128


def _aggord(nag, tdim):
    rows_pw = nag // 32
    nchunks = EPAD // CHE

    @functools.partial(
        pl.kernel, mesh=_mesh,
        compiler_params=pltpu.CompilerParams(needs_layout_passes=False),
        out_type=jax.ShapeDtypeStruct((nag, H), jnp.float32),
        scratch_types=[
            pltpu.VMEM((CHE,), jnp.int32),
            pltpu.VMEM((CHE,), jnp.int32),
            pltpu.VMEM((PEND,), jnp.int32),
            pltpu.VMEM((PEND,), jnp.int32),
            pltpu.VMEM((FL, H), jnp.float32),
            pltpu.VMEM((rows_pw, H), jnp.float32),
            pltpu.SemaphoreType.DMA,
        ],
    )
    def k(y_hbm, src_hbm, dst_hbm, zeros_hbm, out_hbm,
          sv_v, dv_v, psrc_v, pdst_v, rows_v, acc_v, sem):
        c = lax.axis_index("c")
        s = lax.axis_index("s")
        wid = s * 2 + c
        lo = wid * rows_pw
        pltpu.sync_copy(zeros_hbm, acc_v)
        zi = jnp.zeros((16,), jnp.int32)
        for kk in range(PEND // 16):
            psrc_v[pl.ds(kk * 16, 16)] = zi

        def add_edge(r, _):
            dl = pdst_v[pl.ds(r, 16)][0]
            for cc in range(H // 16):
                sl = pl.ds(cc * 16, 16)
                acc_v[dl, sl] = acc_v[dl, sl] + rows_v[r, sl]
            return _

        def flush():
            pltpu.async_copy(y_hbm.at[psrc_v.at[pl.ds(0, FL)]], rows_v, sem).wait()
            lax.fori_loop(0, FL, add_edge, 0)
            for kk in range(PEND // 16 - FL // 16):
                sld = pl.ds(FL + kk * 16, 16)
                sls = pl.ds(kk * 16, 16)
                psrc_v[sls] = psrc_v[sld]
                pdst_v[sls] = pdst_v[sld]

        def group(g, np_):
            sl = pl.ds(g * 16, 16)
            dv = dv_v[sl]
            m = (dv >= lo) & (dv < lo + rows_pw)
            cnt = plsc.all_reduce_population_count(m)[0]

            def with_matches(np_):
                pos = np_ + plsc.cumsum(jnp.where(m, 1, 0)) - 1
                plsc.store_scatter(psrc_v, [pos], sv_v[sl], mask=m)
                plsc.store_scatter(pdst_v, [pos], dv - lo, mask=m)
                np2 = np_ + cnt
                return lax.cond(np2 >= FL, lambda: (flush(), np2 - FL)[1], lambda: np2)

            return lax.cond(cnt > 0, with_matches, lambda n: n, np_)

        def chunk(j, np_):
            pltpu.sync_copy(src_hbm.at[pl.ds(j * CHE, CHE)], sv_v)
            pltpu.sync_copy(dst_hbm.at[pl.ds(j * CHE, CHE)], dv_v)
            return lax.fori_loop(0, CHE // 16, group, np_)

        np_ = lax.fori_loop(0, nchunks, chunk, jnp.int32(0))
        pltpu.async_copy(y_hbm.at[psrc_v.at[pl.ds(0, FL)]], rows_v, sem).wait()
        lax.fori_loop(0, np_, add_edge, 0)
        pltpu.sync_copy(acc_v, out_hbm.at[pl.ds(lo, rows_pw)])

    return k


def _rowscat(npad_in, nag_out, dummy, chunk, nchunks):
    rows_pw = npad_in // 32

    @functools.partial(
        pl.kernel, mesh=_mesh,
        compiler_params=pltpu.CompilerParams(needs_layout_passes=False),
        out_type=jax.ShapeDtypeStruct((nag_out, H), jnp.float32),
        scratch_types=[
            pltpu.VMEM((chunk,), jnp.int32),
            pltpu.VMEM((chunk,), jnp.int32),
            pltpu.VMEM((chunk, H), jnp.float32),
            pltpu.SemaphoreType.DMA,
        ],
    )
    def k(hs_hbm, rank_hbm, out_hbm, rank_v, idx_v, rows_v, sem):
        c = lax.axis_index("c")
        s = lax.axis_index("s")
        wid = s * 2 + c
        base = wid * rows_pw

        def step(j, carry):
            off = base + j * chunk
            pltpu.sync_copy(rank_hbm.at[pl.ds(off, chunk)], rank_v)
            for jj in range(chunk // 16):
                sl = pl.ds(jj * 16, 16)
                idx_v[sl] = jnp.minimum(rank_v[sl], jnp.int32(dummy))
            pltpu.sync_copy(hs_hbm.at[pl.ds(off, chunk)], rows_v)
            pltpu.async_copy(rows_v, out_hbm.at[idx_v], sem).wait()
            return carry

        lax.fori_loop(0, nchunks, step, 0)

    return k


def _remap(npad, kkeep):
    @functools.partial(
        pl.kernel, mesh=_mesh,
        compiler_params=pltpu.CompilerParams(needs_layout_passes=False),
        out_type=(jax.ShapeDtypeStruct((EPAD,), jnp.int32),
                  jax.ShapeDtypeStruct((EPAD,), jnp.int32)),
        scratch_types=[
            pltpu.VMEM((npad,), jnp.int32),
            pltpu.VMEM((EPW,), jnp.int32),
            pltpu.VMEM((EPW,), jnp.int32),
            pltpu.VMEM((EPW,), jnp.int32),
            pltpu.VMEM((EPW,), jnp.int32),
        ],
    )
    def k(src_hbm, dst_hbm, rank_hbm, nsrc_hbm, ndst_hbm, rank_v, src_v, dst_v, osrc_v, odst_v):
        c = lax.axis_index("c")
        s = lax.axis_index("s")
        wid = s * 2 + c
        base = wid * EPW
        pltpu.sync_copy(rank_hbm, rank_v)
        pltpu.sync_copy(src_hbm.at[pl.ds(base, EPW)], src_v)
        pltpu.sync_copy(dst_hbm.at[pl.ds(base, EPW)], dst_v)
        kk = jnp.int32(kkeep)

        def step(j, carry):
            sl = pl.ds(j * 16, 16)
            rs = plsc.load_gather(rank_v, [src_v[sl]])
            rd = plsc.load_gather(rank_v, [dst_v[sl]])
            valid = (rs < kk) & (rd < kk)
            osrc_v[sl] = jnp.where(valid, rs, 0)
            odst_v[sl] = jnp.where(valid, rd, kk)
            return carry

        lax.fori_loop(0, EPW // 16, step, 0)
        pltpu.sync_copy(osrc_v, nsrc_hbm.at[pl.ds(base, EPW)])
        pltpu.sync_copy(odst_v, ndst_hbm.at[pl.ds(base, EPW)])

    return k


def _lsm_body(in_ref, out_ref):
    v = in_ref[...][:NUM_CLASSES]
    m = jnp.max(v, axis=1, keepdims=True)
    lse = jnp.log(jnp.sum(jnp.exp(v - m), axis=1, keepdims=True)) + m
    out_ref[...] = v - lse


_lsm = pl.pallas_call(_lsm_body, out_shape=jax.ShapeDtypeStruct((NUM_CLASSES, H), jnp.float32))


def kernel(x, edge_index, num_target, W1_rel, b1_rel, W1_root, b1_root, pool1_w,
           W2_rel, b2_rel, W2_root, b2_root, pool2_w,
           W3_rel, b3_rel, W3_root, b3_root, pool3_w):
    del num_target
    f32, i32 = jnp.float32, jnp.int32
    src = jnp.concatenate([edge_index[0], jnp.zeros((EPAD - E,), i32)])
    dst = jnp.concatenate([edge_index[1], jnp.full((EPAD - E,), NP1 - 1, i32)])
    xp = jnp.pad(x, ((0, NP1 - N1), (0, 0)))
    zeros1 = jnp.zeros((NP1 // 16, H), f32)
    zeros2 = jnp.zeros((NP2 // 16, H), f32)
    zeros3 = jnp.zeros((NP3 // 16, H), f32)
    def rank_of(key, npad):
        keyr = key.reshape(npad // 128, 1, 128)
        keyt = key.reshape(npad // 32, 32, 1)
        return _rank(npad)(keyr, keyt).reshape(npad)

    z1 = jnp.zeros((NP1 // 32, H), f32)
    plo = _aggord(NP1, H)(xp[:, :128], src, dst, z1)
    phi = _aggord(NP1, H)(xp[:, 128:], src, dst, z1)
    p256 = jnp.concatenate([plo, phi], axis=1)
    hs1, key1 = _combine1(NP1, N1, 1024)(
        p256, xp, W1_rel, W1_root,
        b1_rel.reshape(1, H), b1_root.reshape(1, H), pool1_w.reshape(H, 1))
    rank1 = rank_of(key1, NP1)
    h2in = _rowscat(NP1, NP2, K1, 64, 5)(hs1, rank1)
    src2, dst2 = _remap(NP1, K1)(src, dst, rank1)

    p2 = _aggord(NP2, H)(h2in, src2, dst2, jnp.zeros((NP2 // 32, H), f32))
    hs2, key2 = _combine23(NP2, K1, 1024)(
        p2, h2in, W2_rel, W2_root,
        b2_rel.reshape(1, H), b2_root.reshape(1, H), pool2_w.reshape(H, 1))
    rank2 = rank_of(key2, NP2)
    h3in = _rowscat(NP2, NP3, K2, 128, 2)(hs2, rank2)
    src3, dst3 = _remap(NP2, K2)(src2, dst2, rank2)

    p3 = _aggord(NP3, H)(h3in, src3, dst3, jnp.zeros((NP3 // 32, H), f32))
    hs3, key3 = _combine23(NP3, K2, 832)(
        p3, h3in, W3_rel, W3_root,
        b3_rel.reshape(1, H), b3_root.reshape(1, H), pool3_w.reshape(H, 1))
    rank3 = rank_of(key3, NP3)
    out16 = _rowscat(NP3, 16, 15, 16, 13)(hs3, rank3)
    return _lsm(out16)


# speedup vs baseline: 3.2195x; 3.2195x over previous
import functools
import jax
import jax.numpy as jnp
from jax import lax
from jax.experimental import pallas as pl
from jax.experimental.pallas import tpu as pltpu
from jax.experimental.pallas import tpu_sc as plsc

N1, E, D, H = 10000, 160000, 256, 128
NUM_CLASSES = 10
K1, K2 = 8000, 6400
NP1, NP2, NP3 = 10240, 8192, 6656
EPAD = 163840
EPW = EPAD // 32
CH = 128
IMIN = -2147483648

_mesh = plsc.VectorSubcoreMesh(core_axis_name="c", subcore_axis_name="s")


def _key_from_score(s, rows, n_real):
    b = lax.bitcast_convert_type(s, jnp.int32)
    b = jnp.where(b == IMIN, 0, b)
    key = jnp.where(b < 0, b ^ 0x7FFFFFFF, b)
    return jnp.where(rows < n_real, key, IMIN)


def _score_tail(h, w, br, n_real, hs_ref, key_ref):
    wn = jnp.sqrt(jnp.sum(w * w))
    arg = lax.dot_general(h, w, (((1,), (0,)), ((), ())),
                          preferred_element_type=jnp.float32) / wn
    s = jnp.tanh(arg)
    hs_ref[...] = h * s
    rows = pl.program_id(0) * br + lax.broadcasted_iota(jnp.int32, (br, 1), 0)
    key_ref[...] = _key_from_score(s, rows, n_real)


_DN = (((1,), (1,)), ((), ()))


def _combine1_body(n_real, br, p_ref, x_ref, wrel_ref, wroot_ref,
                   ba_ref, bb_ref, w_ref, hs_ref, key_ref):
    a = lax.dot_general(p_ref[...], wrel_ref[...], _DN,
                        preferred_element_type=jnp.float32)
    xr = lax.dot_general(x_ref[...], wroot_ref[...], _DN,
                         preferred_element_type=jnp.float32)
    h = jnp.maximum((a + ba_ref[...]) + (xr + bb_ref[...]), 0.0)
    _score_tail(h, w_ref[...], br, n_real, hs_ref, key_ref)


def _combine1(npad, n_real, br):
    return pl.pallas_call(
        functools.partial(_combine1_body, n_real, br),
        grid=(npad // br,),
        in_specs=[
            pl.BlockSpec((br, D), lambda i: (i, 0)),
            pl.BlockSpec((br, D), lambda i: (i, 0)),
            pl.BlockSpec((H, D), lambda i: (0, 0)),
            pl.BlockSpec((H, D), lambda i: (0, 0)),
            pl.BlockSpec((1, H), lambda i: (0, 0)),
            pl.BlockSpec((1, H), lambda i: (0, 0)),
            pl.BlockSpec((H, 1), lambda i: (0, 0)),
        ],
        out_specs=[
            pl.BlockSpec((br, H), lambda i: (i, 0)),
            pl.BlockSpec((br, 1), lambda i: (i, 0)),
        ],
        out_shape=[
            jax.ShapeDtypeStruct((npad, H), jnp.float32),
            jax.ShapeDtypeStruct((npad, 1), jnp.int32),
        ],
    )


def _combine23_body(n_real, br, p_ref, hin_ref, wrel_ref, wroot_ref,
                    ba_ref, bb_ref, w_ref, hs_ref, key_ref):
    a = lax.dot_general(p_ref[...], wrel_ref[...], _DN,
                        preferred_element_type=jnp.float32)
    xr = lax.dot_general(hin_ref[...], wroot_ref[...], _DN,
                         preferred_element_type=jnp.float32)
    h = jnp.maximum((a + ba_ref[...]) + (xr + bb_ref[...]), 0.0)
    _score_tail(h, w_ref[...], br, n_real, hs_ref, key_ref)


def _combine23(npad, n_real, br):
    return pl.pallas_call(
        functools.partial(_combine23_body, n_real, br),
        grid=(npad // br,),
        in_specs=[
            pl.BlockSpec((br, H), lambda i: (i, 0)),
            pl.BlockSpec((br, H), lambda i: (i, 0)),
            pl.BlockSpec((H, H), lambda i: (0, 0)),
            pl.BlockSpec((H, H), lambda i: (0, 0)),
            pl.BlockSpec((1, H), lambda i: (0, 0)),
            pl.BlockSpec((1, H), lambda i: (0, 0)),
            pl.BlockSpec((H, 1), lambda i: (0, 0)),
        ],
        out_specs=[
            pl.BlockSpec((br, H), lambda i: (i, 0)),
            pl.BlockSpec((br, 1), lambda i: (i, 0)),
        ],
        out_shape=[
            jax.ShapeDtypeStruct((npad, H), jnp.float32),
            jax.ShapeDtypeStruct((npad, 1), jnp.int32),
        ],
    )


def _rank_body(cchunks, ki_ref, kt_ref, rank_ref):
    bi = pl.program_id(0)
    ki = ki_ref[0]
    iglob = bi * 128 + lax.broadcasted_iota(jnp.int32, (1, 128), 1)

    def step(c, acc):
        kj = kt_ref[c]
        jglob = c * 32 + lax.broadcasted_iota(jnp.int32, (32, 1), 0)
        jlt = jglob < iglob
        ge = jnp.where(kj >= ki, jnp.int32(1), jnp.int32(0))
        gt = jnp.where(kj > ki, jnp.int32(1), jnp.int32(0))
        return acc + jnp.where(jlt, ge, gt)

    acc = lax.fori_loop(0, cchunks, step, jnp.zeros((32, 128), jnp.int32))
    rank_ref[0] = jnp.sum(acc, axis=0, keepdims=True)


def _rank(npad):
    rrows, cchunks = npad // 128, npad // 32
    return pl.pallas_call(
        functools.partial(_rank_body, cchunks),
        grid=(rrows,),
        in_specs=[
            pl.BlockSpec((1, 1, 128), lambda i: (i, 0, 0)),
            pl.BlockSpec((cchunks, 32, 1), lambda i: (0, 0, 0)),
        ],
        out_specs=pl.BlockSpec((1, 1, 128), lambda i: (i, 0, 0)),
        out_shape=jax.ShapeDtypeStruct((rrows, 1, 128), jnp.int32),
    )


CHE = 4096
PEND = 256
FL = 128


def _aggord(nag, dummy):
    rows_pw = nag // 32
    nchunks = EPAD // CHE

    @functools.partial(
        pl.kernel, mesh=_mesh,
        compiler_params=pltpu.CompilerParams(needs_layout_passes=False),
        out_type=jax.ShapeDtypeStruct((nag, H), jnp.float32),
        scratch_types=[
            pltpu.VMEM((CHE,), jnp.int32),
            pltpu.VMEM((CHE,), jnp.int32),
            pltpu.VMEM((PEND,), jnp.int32),
            pltpu.VMEM((PEND,), jnp.int32),
            pltpu.VMEM((FL, H), jnp.float32),
            pltpu.VMEM((rows_pw, H), jnp.float32),
            pltpu.SemaphoreType.DMA,
        ],
    )
    def k(y_hbm, src_hbm, dst_hbm, zeros_hbm, out_hbm,
          sv_v, dv_v, psrc_v, pdst_v, rows_v, acc_v, sem):
        c = lax.axis_index("c")
        s = lax.axis_index("s")
        wid = s * 2 + c
        lo = wid * rows_pw
        pltpu.sync_copy(zeros_hbm, acc_v)
        zi = jnp.zeros((16,), jnp.int32)
        for kk in range(PEND // 16):
            psrc_v[pl.ds(kk * 16, 16)] = zi

        def add_edge(r, _):
            dl = pdst_v[pl.ds(r, 16)][0]
            for cc in range(H // 16):
                sl = pl.ds(cc * 16, 16)
                acc_v[dl, sl] = acc_v[dl, sl] + rows_v[r, sl]
            return _

        def flush():
            pltpu.async_copy(y_hbm.at[psrc_v.at[pl.ds(0, FL)]], rows_v, sem).wait()
            lax.fori_loop(0, FL, add_edge, 0)
            for kk in range(PEND // 16 - FL // 16):
                sld = pl.ds(FL + kk * 16, 16)
                sls = pl.ds(kk * 16, 16)
                psrc_v[sls] = psrc_v[sld]
                pdst_v[sls] = pdst_v[sld]

        def group(g, np_):
            sl = pl.ds(g * 16, 16)
            dv = dv_v[sl]
            m = (dv >= lo) & (dv < lo + rows_pw) & (dv != dummy)
            cnt = plsc.all_reduce_population_count(m)[0]

            def with_matches(np_):
                pos = np_ + plsc.cumsum(jnp.where(m, 1, 0)) - 1
                plsc.store_scatter(psrc_v, [pos], sv_v[sl], mask=m)
                plsc.store_scatter(pdst_v, [pos], dv - lo, mask=m)
                np2 = np_ + cnt
                return lax.cond(np2 >= FL, lambda: (flush(), np2 - FL)[1], lambda: np2)

            return lax.cond(cnt > 0, with_matches, lambda n: n, np_)

        def chunk(j, np_):
            pltpu.sync_copy(src_hbm.at[pl.ds(j * CHE, CHE)], sv_v)
            pltpu.sync_copy(dst_hbm.at[pl.ds(j * CHE, CHE)], dv_v)
            return lax.fori_loop(0, CHE // 16, group, np_)

        np_ = lax.fori_loop(0, nchunks, chunk, jnp.int32(0))
        pltpu.async_copy(y_hbm.at[psrc_v.at[pl.ds(0, FL)]], rows_v, sem).wait()
        lax.fori_loop(0, np_, add_edge, 0)
        pltpu.sync_copy(acc_v, out_hbm.at[pl.ds(lo, rows_pw)])

    return k


def _rowscat(npad_in, nag_out, dummy, chunk, nchunks):
    rows_pw = npad_in // 32

    @functools.partial(
        pl.kernel, mesh=_mesh,
        compiler_params=pltpu.CompilerParams(needs_layout_passes=False),
        out_type=jax.ShapeDtypeStruct((nag_out, H), jnp.float32),
        scratch_types=[
            pltpu.VMEM((chunk,), jnp.int32),
            pltpu.VMEM((chunk,), jnp.int32),
            pltpu.VMEM((chunk, H), jnp.float32),
            pltpu.SemaphoreType.DMA,
        ],
    )
    def k(hs_hbm, rank_hbm, out_hbm, rank_v, idx_v, rows_v, sem):
        c = lax.axis_index("c")
        s = lax.axis_index("s")
        wid = s * 2 + c
        base = wid * rows_pw

        def step(j, carry):
            off = base + j * chunk
            pltpu.sync_copy(rank_hbm.at[pl.ds(off, chunk)], rank_v)
            for jj in range(chunk // 16):
                sl = pl.ds(jj * 16, 16)
                idx_v[sl] = jnp.minimum(rank_v[sl], jnp.int32(dummy))
            pltpu.sync_copy(hs_hbm.at[pl.ds(off, chunk)], rows_v)
            pltpu.async_copy(rows_v, out_hbm.at[idx_v], sem).wait()
            return carry

        lax.fori_loop(0, nchunks, step, 0)

    return k


def _remap(npad, kkeep):
    @functools.partial(
        pl.kernel, mesh=_mesh,
        compiler_params=pltpu.CompilerParams(needs_layout_passes=False),
        out_type=(jax.ShapeDtypeStruct((EPAD,), jnp.int32),
                  jax.ShapeDtypeStruct((EPAD,), jnp.int32)),
        scratch_types=[
            pltpu.VMEM((npad,), jnp.int32),
            pltpu.VMEM((EPW,), jnp.int32),
            pltpu.VMEM((EPW,), jnp.int32),
            pltpu.VMEM((EPW,), jnp.int32),
            pltpu.VMEM((EPW,), jnp.int32),
        ],
    )
    def k(src_hbm, dst_hbm, rank_hbm, nsrc_hbm, ndst_hbm, rank_v, src_v, dst_v, osrc_v, odst_v):
        c = lax.axis_index("c")
        s = lax.axis_index("s")
        wid = s * 2 + c
        base = wid * EPW
        pltpu.sync_copy(rank_hbm, rank_v)
        pltpu.sync_copy(src_hbm.at[pl.ds(base, EPW)], src_v)
        pltpu.sync_copy(dst_hbm.at[pl.ds(base, EPW)], dst_v)
        kk = jnp.int32(kkeep)

        def step(j, carry):
            sl = pl.ds(j * 16, 16)
            rs = plsc.load_gather(rank_v, [src_v[sl]])
            rd = plsc.load_gather(rank_v, [dst_v[sl]])
            valid = (rs < kk) & (rd < kk)
            osrc_v[sl] = jnp.where(valid, rs, 0)
            odst_v[sl] = jnp.where(valid, rd, kk)
            return carry

        lax.fori_loop(0, EPW // 16, step, 0)
        pltpu.sync_copy(osrc_v, nsrc_hbm.at[pl.ds(base, EPW)])
        pltpu.sync_copy(odst_v, ndst_hbm.at[pl.ds(base, EPW)])

    return k


def _lsm_body(in_ref, out_ref):
    v = in_ref[...][:NUM_CLASSES]
    m = jnp.max(v, axis=1, keepdims=True)
    lse = jnp.log(jnp.sum(jnp.exp(v - m), axis=1, keepdims=True)) + m
    out_ref[...] = v - lse


_lsm = pl.pallas_call(_lsm_body, out_shape=jax.ShapeDtypeStruct((NUM_CLASSES, H), jnp.float32))


def kernel(x, edge_index, num_target, W1_rel, b1_rel, W1_root, b1_root, pool1_w,
           W2_rel, b2_rel, W2_root, b2_root, pool2_w,
           W3_rel, b3_rel, W3_root, b3_root, pool3_w):
    del num_target
    f32, i32 = jnp.float32, jnp.int32
    src = jnp.concatenate([edge_index[0], jnp.zeros((EPAD - E,), i32)])
    dst = jnp.concatenate([edge_index[1], jnp.full((EPAD - E,), NP1 - 1, i32)])
    xp = jnp.pad(x, ((0, NP1 - N1), (0, 0)))
    zeros1 = jnp.zeros((NP1 // 16, H), f32)
    zeros2 = jnp.zeros((NP2 // 16, H), f32)
    zeros3 = jnp.zeros((NP3 // 16, H), f32)
    def rank_of(key, npad):
        keyr = key.reshape(npad // 128, 1, 128)
        keyt = key.reshape(npad // 32, 32, 1)
        return _rank(npad)(keyr, keyt).reshape(npad)

    z1 = jnp.zeros((NP1 // 32, H), f32)
    plo = _aggord(NP1, NP1 - 1)(xp[:, :128], src, dst, z1)
    phi = _aggord(NP1, NP1 - 1)(xp[:, 128:], src, dst, z1)
    p256 = jnp.concatenate([plo, phi], axis=1)
    hs1, key1 = _combine1(NP1, N1, 1024)(
        p256, xp, W1_rel, W1_root,
        b1_rel.reshape(1, H), b1_root.reshape(1, H), pool1_w.reshape(H, 1))
    rank1 = rank_of(key1, NP1)
    h2in = _rowscat(NP1, NP2, K1, 64, 5)(hs1, rank1)
    src2, dst2 = _remap(NP1, K1)(src, dst, rank1)

    p2 = _aggord(NP2, K1)(h2in, src2, dst2, jnp.zeros((NP2 // 32, H), f32))
    hs2, key2 = _combine23(NP2, K1, 1024)(
        p2, h2in, W2_rel, W2_root,
        b2_rel.reshape(1, H), b2_root.reshape(1, H), pool2_w.reshape(H, 1))
    rank2 = rank_of(key2, NP2)
    h3in = _rowscat(NP2, NP3, K2, 128, 2)(hs2, rank2)
    src3, dst3 = _remap(NP2, K2)(src2, dst2, rank2)

    p3 = _aggord(NP3, K2)(h3in, src3, dst3, jnp.zeros((NP3 // 32, H), f32))
    hs3, key3 = _combine23(NP3, K2, 832)(
        p3, h3in, W3_rel, W3_root,
        b3_rel.reshape(1, H), b3_root.reshape(1, H), pool3_w.reshape(H, 1))
    rank3 = rank_of(key3, NP3)
    out16 = _rowscat(NP3, 16, 15, 16, 13)(hs3, rank3)
    return _lsm(out16)


# speedup vs baseline: 3.4594x; 1.0745x over previous
import functools
import jax
import jax.numpy as jnp
from jax import lax
from jax.experimental import pallas as pl
from jax.experimental.pallas import tpu as pltpu
from jax.experimental.pallas import tpu_sc as plsc

N1, E, D, H = 10000, 160000, 256, 128
NUM_CLASSES = 10
K1, K2 = 8000, 6400
NP1, NP2, NP3 = 10240, 8192, 6656
EPAD = 163840
EPW = EPAD // 32
CH = 128
IMIN = -2147483648

_mesh = plsc.VectorSubcoreMesh(core_axis_name="c", subcore_axis_name="s")


def _key_from_score(s, rows, n_real):
    b = lax.bitcast_convert_type(s, jnp.int32)
    b = jnp.where(b == IMIN, 0, b)
    key = jnp.where(b < 0, b ^ 0x7FFFFFFF, b)
    return jnp.where(rows < n_real, key, IMIN)


def _score_tail(h, w, br, n_real, hs_ref, key_ref):
    wn = jnp.sqrt(jnp.sum(w * w))
    arg = lax.dot_general(h, w, (((1,), (0,)), ((), ())),
                          preferred_element_type=jnp.float32) / wn
    s = jnp.tanh(arg)
    hs_ref[...] = h * s
    rows = pl.program_id(0) * br + lax.broadcasted_iota(jnp.int32, (br, 1), 0)
    key_ref[...] = _key_from_score(s, rows, n_real)


_DN = (((1,), (1,)), ((), ()))


def _combine1_body(n_real, br, p_ref, x_ref, wrel_ref, wroot_ref,
                   ba_ref, bb_ref, w_ref, hs_ref, key_ref):
    a = lax.dot_general(p_ref[...], wrel_ref[...], _DN,
                        preferred_element_type=jnp.float32)
    xr = lax.dot_general(x_ref[...], wroot_ref[...], _DN,
                         preferred_element_type=jnp.float32)
    h = jnp.maximum((a + ba_ref[...]) + (xr + bb_ref[...]), 0.0)
    _score_tail(h, w_ref[...], br, n_real, hs_ref, key_ref)


def _combine1(npad, n_real, br):
    return pl.pallas_call(
        functools.partial(_combine1_body, n_real, br),
        grid=(npad // br,),
        in_specs=[
            pl.BlockSpec((br, D), lambda i: (i, 0)),
            pl.BlockSpec((br, D), lambda i: (i, 0)),
            pl.BlockSpec((H, D), lambda i: (0, 0)),
            pl.BlockSpec((H, D), lambda i: (0, 0)),
            pl.BlockSpec((1, H), lambda i: (0, 0)),
            pl.BlockSpec((1, H), lambda i: (0, 0)),
            pl.BlockSpec((H, 1), lambda i: (0, 0)),
        ],
        out_specs=[
            pl.BlockSpec((br, H), lambda i: (i, 0)),
            pl.BlockSpec((br, 1), lambda i: (i, 0)),
        ],
        out_shape=[
            jax.ShapeDtypeStruct((npad, H), jnp.float32),
            jax.ShapeDtypeStruct((npad, 1), jnp.int32),
        ],
    )


def _combine23_body(n_real, br, p_ref, hin_ref, wrel_ref, wroot_ref,
                    ba_ref, bb_ref, w_ref, hs_ref, key_ref):
    a = lax.dot_general(p_ref[...], wrel_ref[...], _DN,
                        preferred_element_type=jnp.float32)
    xr = lax.dot_general(hin_ref[...], wroot_ref[...], _DN,
                         preferred_element_type=jnp.float32)
    h = jnp.maximum((a + ba_ref[...]) + (xr + bb_ref[...]), 0.0)
    _score_tail(h, w_ref[...], br, n_real, hs_ref, key_ref)


def _combine23(npad, n_real, br):
    return pl.pallas_call(
        functools.partial(_combine23_body, n_real, br),
        grid=(npad // br,),
        in_specs=[
            pl.BlockSpec((br, H), lambda i: (i, 0)),
            pl.BlockSpec((br, H), lambda i: (i, 0)),
            pl.BlockSpec((H, H), lambda i: (0, 0)),
            pl.BlockSpec((H, H), lambda i: (0, 0)),
            pl.BlockSpec((1, H), lambda i: (0, 0)),
            pl.BlockSpec((1, H), lambda i: (0, 0)),
            pl.BlockSpec((H, 1), lambda i: (0, 0)),
        ],
        out_specs=[
            pl.BlockSpec((br, H), lambda i: (i, 0)),
            pl.BlockSpec((br, 1), lambda i: (i, 0)),
        ],
        out_shape=[
            jax.ShapeDtypeStruct((npad, H), jnp.float32),
            jax.ShapeDtypeStruct((npad, 1), jnp.int32),
        ],
    )


def _rank_body(cchunks, ki_ref, kt_ref, rank_ref):
    bi = pl.program_id(0)
    ki = ki_ref[0]
    iglob = bi * 128 + lax.broadcasted_iota(jnp.int32, (1, 128), 1)

    def step(c, acc):
        kj = kt_ref[c]
        jglob = c * 32 + lax.broadcasted_iota(jnp.int32, (32, 1), 0)
        jlt = jglob < iglob
        ge = jnp.where(kj >= ki, jnp.int32(1), jnp.int32(0))
        gt = jnp.where(kj > ki, jnp.int32(1), jnp.int32(0))
        return acc + jnp.where(jlt, ge, gt)

    acc = lax.fori_loop(0, cchunks, step, jnp.zeros((32, 128), jnp.int32))
    rank_ref[0] = jnp.sum(acc, axis=0, keepdims=True)


def _rank(npad):
    rrows, cchunks = npad // 128, npad // 32
    return pl.pallas_call(
        functools.partial(_rank_body, cchunks),
        grid=(rrows,),
        in_specs=[
            pl.BlockSpec((1, 1, 128), lambda i: (i, 0, 0)),
            pl.BlockSpec((cchunks, 32, 1), lambda i: (0, 0, 0)),
        ],
        out_specs=pl.BlockSpec((1, 1, 128), lambda i: (i, 0, 0)),
        out_shape=jax.ShapeDtypeStruct((rrows, 1, 128), jnp.int32),
    )


CHE = 4096
PEND = 256
FL = 128


def _aggord(nag, dummy):
    rows_pw = nag // 32
    nchunks = EPAD // CHE

    @functools.partial(
        pl.kernel, mesh=_mesh,
        compiler_params=pltpu.CompilerParams(needs_layout_passes=False),
        out_type=jax.ShapeDtypeStruct((nag, H), jnp.float32),
        scratch_types=[
            pltpu.VMEM((CHE,), jnp.int32),
            pltpu.VMEM((CHE,), jnp.int32),
            pltpu.VMEM((PEND,), jnp.int32),
            pltpu.VMEM((PEND,), jnp.int32),
            pltpu.VMEM((FL, H), jnp.float32),
            pltpu.VMEM((rows_pw, H), jnp.float32),
            pltpu.SemaphoreType.DMA,
        ],
    )
    def k(y_hbm, src_hbm, dst_hbm, zeros_hbm, out_hbm,
          sv_v, dv_v, psrc_v, pdst_v, rows_v, acc_v, sem):
        c = lax.axis_index("c")
        s = lax.axis_index("s")
        wid = s * 2 + c
        lo = wid * rows_pw
        pltpu.sync_copy(zeros_hbm, acc_v)
        zi = jnp.zeros((16,), jnp.int32)
        for kk in range(PEND // 16):
            psrc_v[pl.ds(kk * 16, 16)] = zi

        def add_edge(r, _):
            dl = pdst_v[pl.ds(r, 16)][0]
            for cc in range(H // 16):
                sl = pl.ds(cc * 16, 16)
                acc_v[dl, sl] = acc_v[dl, sl] + rows_v[r, sl]
            return _

        def flush():
            pltpu.async_copy(y_hbm.at[psrc_v.at[pl.ds(0, FL)]], rows_v, sem).wait()
            lax.fori_loop(0, FL, add_edge, 0)
            for kk in range(PEND // 16 - FL // 16):
                sld = pl.ds(FL + kk * 16, 16)
                sls = pl.ds(kk * 16, 16)
                psrc_v[sls] = psrc_v[sld]
                pdst_v[sls] = pdst_v[sld]

        def group(g, np_):
            sl = pl.ds(g * 16, 16)
            dv = dv_v[sl]
            m = (dv >= lo) & (dv < lo + rows_pw) & (dv != dummy)
            cnt = plsc.all_reduce_population_count(m)[0]

            def with_matches(np_):
                pos = np_ + plsc.cumsum(jnp.where(m, 1, 0)) - 1
                plsc.store_scatter(psrc_v, [pos], sv_v[sl], mask=m)
                plsc.store_scatter(pdst_v, [pos], dv - lo, mask=m)
                np2 = np_ + cnt
                return lax.cond(np2 >= FL, lambda: (flush(), np2 - FL)[1], lambda: np2)

            return lax.cond(cnt > 0, with_matches, lambda n: n, np_)

        def chunk(j, np_):
            pltpu.sync_copy(src_hbm.at[pl.ds(j * CHE, CHE)], sv_v)
            pltpu.sync_copy(dst_hbm.at[pl.ds(j * CHE, CHE)], dv_v)
            return lax.fori_loop(0, CHE // 16, group, np_)

        np_ = lax.fori_loop(0, nchunks, chunk, jnp.int32(0))
        pltpu.async_copy(y_hbm.at[psrc_v.at[pl.ds(0, FL)]], rows_v, sem).wait()
        lax.fori_loop(0, np_, add_edge, 0)
        pltpu.sync_copy(acc_v, out_hbm.at[pl.ds(lo, rows_pw)])

    return k


def _aggord2(nag, dummy):
    rows_pw = nag // 32
    nchunks = EPAD // CHE
    FL2 = 64

    @functools.partial(
        pl.kernel, mesh=_mesh,
        compiler_params=pltpu.CompilerParams(needs_layout_passes=False),
        out_type=jax.ShapeDtypeStruct((nag, D), jnp.float32),
        scratch_types=[
            pltpu.VMEM((CHE,), jnp.int32),
            pltpu.VMEM((CHE,), jnp.int32),
            pltpu.VMEM((PEND,), jnp.int32),
            pltpu.VMEM((PEND,), jnp.int32),
            pltpu.VMEM((64, H), jnp.float32),
            pltpu.VMEM((64, H), jnp.float32),
            pltpu.VMEM((rows_pw, D), jnp.float32),
            pltpu.SemaphoreType.DMA,
        ],
    )
    def k(ylo_hbm, yhi_hbm, src_hbm, dst_hbm, zeros_hbm, out_hbm,
          sv_v, dv_v, psrc_v, pdst_v, rlo_v, rhi_v, acc_v, sem):
        c = lax.axis_index("c")
        s = lax.axis_index("s")
        wid = s * 2 + c
        lo = wid * rows_pw
        pltpu.sync_copy(zeros_hbm, acc_v)
        zi = jnp.zeros((16,), jnp.int32)
        for kk in range(PEND // 16):
            psrc_v[pl.ds(kk * 16, 16)] = zi

        def add_edge(r, _):
            dl = pdst_v[pl.ds(r, 16)][0]
            for cc in range(H // 16):
                sl = pl.ds(cc * 16, 16)
                acc_v[dl, sl] = acc_v[dl, sl] + rlo_v[r, sl]
                sl2 = pl.ds(H + cc * 16, 16)
                acc_v[dl, sl2] = acc_v[dl, sl2] + rhi_v[r, sl]
            return _

        def flush():
            pltpu.async_copy(ylo_hbm.at[psrc_v.at[pl.ds(0, FL2)]], rlo_v, sem).wait()
            pltpu.async_copy(yhi_hbm.at[psrc_v.at[pl.ds(0, FL2)]], rhi_v, sem).wait()
            lax.fori_loop(0, FL2, add_edge, 0)
            for kk in range(PEND // 16 - FL2 // 16):
                sld = pl.ds(FL2 + kk * 16, 16)
                sls = pl.ds(kk * 16, 16)
                psrc_v[sls] = psrc_v[sld]
                pdst_v[sls] = pdst_v[sld]

        def group(g, np_):
            sl = pl.ds(g * 16, 16)
            dv = dv_v[sl]
            m = (dv >= lo) & (dv < lo + rows_pw) & (dv != dummy)
            cnt = plsc.all_reduce_population_count(m)[0]

            def with_matches(np_):
                pos = np_ + plsc.cumsum(jnp.where(m, 1, 0)) - 1
                plsc.store_scatter(psrc_v, [pos], sv_v[sl], mask=m)
                plsc.store_scatter(pdst_v, [pos], dv - lo, mask=m)
                np2 = np_ + cnt
                return lax.cond(np2 >= FL2, lambda: (flush(), np2 - FL2)[1], lambda: np2)

            return lax.cond(cnt > 0, with_matches, lambda n: n, np_)

        def chunk(j, np_):
            pltpu.sync_copy(src_hbm.at[pl.ds(j * CHE, CHE)], sv_v)
            pltpu.sync_copy(dst_hbm.at[pl.ds(j * CHE, CHE)], dv_v)
            return lax.fori_loop(0, CHE // 16, group, np_)

        np_ = lax.fori_loop(0, nchunks, chunk, jnp.int32(0))
        pltpu.async_copy(ylo_hbm.at[psrc_v.at[pl.ds(0, FL2)]], rlo_v, sem).wait()
        pltpu.async_copy(yhi_hbm.at[psrc_v.at[pl.ds(0, FL2)]], rhi_v, sem).wait()
        lax.fori_loop(0, np_, add_edge, 0)
        pltpu.sync_copy(acc_v, out_hbm.at[pl.ds(lo, rows_pw)])

    return k


def _rowscat(npad_in, nag_out, dummy, chunk, nchunks):
    rows_pw = npad_in // 32

    @functools.partial(
        pl.kernel, mesh=_mesh,
        compiler_params=pltpu.CompilerParams(needs_layout_passes=False),
        out_type=jax.ShapeDtypeStruct((nag_out, H), jnp.float32),
        scratch_types=[
            pltpu.VMEM((chunk,), jnp.int32),
            pltpu.VMEM((chunk,), jnp.int32),
            pltpu.VMEM((chunk, H), jnp.float32),
            pltpu.SemaphoreType.DMA,
        ],
    )
    def k(hs_hbm, rank_hbm, out_hbm, rank_v, idx_v, rows_v, sem):
        c = lax.axis_index("c")
        s = lax.axis_index("s")
        wid = s * 2 + c
        base = wid * rows_pw

        def step(j, carry):
            off = base + j * chunk
            pltpu.sync_copy(rank_hbm.at[pl.ds(off, chunk)], rank_v)
            for jj in range(chunk // 16):
                sl = pl.ds(jj * 16, 16)
                idx_v[sl] = jnp.minimum(rank_v[sl], jnp.int32(dummy))
            pltpu.sync_copy(hs_hbm.at[pl.ds(off, chunk)], rows_v)
            pltpu.async_copy(rows_v, out_hbm.at[idx_v], sem).wait()
            return carry

        lax.fori_loop(0, nchunks, step, 0)

    return k


def _remap(npad, kkeep):
    @functools.partial(
        pl.kernel, mesh=_mesh,
        compiler_params=pltpu.CompilerParams(needs_layout_passes=False),
        out_type=(jax.ShapeDtypeStruct((EPAD,), jnp.int32),
                  jax.ShapeDtypeStruct((EPAD,), jnp.int32)),
        scratch_types=[
            pltpu.VMEM((npad,), jnp.int32),
            pltpu.VMEM((EPW,), jnp.int32),
            pltpu.VMEM((EPW,), jnp.int32),
            pltpu.VMEM((EPW,), jnp.int32),
            pltpu.VMEM((EPW,), jnp.int32),
        ],
    )
    def k(src_hbm, dst_hbm, rank_hbm, nsrc_hbm, ndst_hbm, rank_v, src_v, dst_v, osrc_v, odst_v):
        c = lax.axis_index("c")
        s = lax.axis_index("s")
        wid = s * 2 + c
        base = wid * EPW
        pltpu.sync_copy(rank_hbm, rank_v)
        pltpu.sync_copy(src_hbm.at[pl.ds(base, EPW)], src_v)
        pltpu.sync_copy(dst_hbm.at[pl.ds(base, EPW)], dst_v)
        kk = jnp.int32(kkeep)

        def step(j, carry):
            sl = pl.ds(j * 16, 16)
            rs = plsc.load_gather(rank_v, [src_v[sl]])
            rd = plsc.load_gather(rank_v, [dst_v[sl]])
            valid = (rs < kk) & (rd < kk)
            osrc_v[sl] = jnp.where(valid, rs, 0)
            odst_v[sl] = jnp.where(valid, rd, kk)
            return carry

        lax.fori_loop(0, EPW // 16, step, 0)
        pltpu.sync_copy(osrc_v, nsrc_hbm.at[pl.ds(base, EPW)])
        pltpu.sync_copy(odst_v, ndst_hbm.at[pl.ds(base, EPW)])

    return k


def _lsm_body(in_ref, out_ref):
    v = in_ref[...][:NUM_CLASSES]
    m = jnp.max(v, axis=1, keepdims=True)
    lse = jnp.log(jnp.sum(jnp.exp(v - m), axis=1, keepdims=True)) + m
    out_ref[...] = v - lse


_lsm = pl.pallas_call(_lsm_body, out_shape=jax.ShapeDtypeStruct((NUM_CLASSES, H), jnp.float32))


def kernel(x, edge_index, num_target, W1_rel, b1_rel, W1_root, b1_root, pool1_w,
           W2_rel, b2_rel, W2_root, b2_root, pool2_w,
           W3_rel, b3_rel, W3_root, b3_root, pool3_w):
    del num_target
    f32, i32 = jnp.float32, jnp.int32
    src = jnp.concatenate([edge_index[0], jnp.zeros((EPAD - E,), i32)])
    dst = jnp.concatenate([edge_index[1], jnp.full((EPAD - E,), NP1 - 1, i32)])
    xp = jnp.pad(x, ((0, NP1 - N1), (0, 0)))
    zeros1 = jnp.zeros((NP1 // 16, H), f32)
    zeros2 = jnp.zeros((NP2 // 16, H), f32)
    zeros3 = jnp.zeros((NP3 // 16, H), f32)
    def rank_of(key, npad):
        keyr = key.reshape(npad // 128, 1, 128)
        keyt = key.reshape(npad // 32, 32, 1)
        return _rank(npad)(keyr, keyt).reshape(npad)

    z1 = jnp.zeros((NP1 // 32, H), f32)
    z256 = jnp.zeros((NP1 // 32, D), f32)
    p256 = _aggord2(NP1, NP1 - 1)(xp[:, :128], xp[:, 128:], src, dst, z256)
    hs1, key1 = _combine1(NP1, N1, 1024)(
        p256, xp, W1_rel, W1_root,
        b1_rel.reshape(1, H), b1_root.reshape(1, H), pool1_w.reshape(H, 1))
    rank1 = rank_of(key1, NP1)
    h2in = _rowscat(NP1, NP2, K1, 64, 5)(hs1, rank1)
    src2, dst2 = _remap(NP1, K1)(src, dst, rank1)

    p2 = _aggord(NP2, K1)(h2in, src2, dst2, jnp.zeros((NP2 // 32, H), f32))
    hs2, key2 = _combine23(NP2, K1, 1024)(
        p2, h2in, W2_rel, W2_root,
        b2_rel.reshape(1, H), b2_root.reshape(1, H), pool2_w.reshape(H, 1))
    rank2 = rank_of(key2, NP2)
    h3in = _rowscat(NP2, NP3, K2, 128, 2)(hs2, rank2)
    src3, dst3 = _remap(NP2, K2)(src2, dst2, rank2)

    p3 = _aggord(NP3, K2)(h3in, src3, dst3, jnp.zeros((NP3 // 32, H), f32))
    hs3, key3 = _combine23(NP3, K2, 832)(
        p3, h3in, W3_rel, W3_root,
        b3_rel.reshape(1, H), b3_root.reshape(1, H), pool3_w.reshape(H, 1))
    rank3 = rank_of(key3, NP3)
    out16 = _rowscat(NP3, 16, 15, 16, 13)(hs3, rank3)
    return _lsm(out16)
